# baseline (speedup 1.0000x reference)
import functools

import jax
import jax.numpy as jnp
from jax import lax
from jax.experimental import pallas as pl
from jax.experimental.pallas import tpu as pltpu
from jax.experimental.pallas import tpu_sc as plsc

BATCH = 16384
VOCAB_SIZE = 1000000
EMBED = 32
LANES = 16
NUM_CORES = 2
NUM_SUBCORES = 16
NUM_WORKERS = NUM_CORES * NUM_SUBCORES
B_PER_W = BATCH // NUM_WORKERS
WAVE = 4
NWAVES = B_PER_W // WAVE
SLOTS = 2 * WAVE


@functools.partial(
    pl.kernel,
    mesh=plsc.VectorSubcoreMesh(core_axis_name="c", subcore_axis_name="s"),
    out_type=jax.ShapeDtypeStruct((BATCH,), jnp.float32),
    compiler_params=pltpu.CompilerParams(needs_layout_passes=False),
    scratch_types=[
        pltpu.VMEM((B_PER_W + LANES,), jnp.int32),
        pltpu.VMEM((B_PER_W + LANES,), jnp.int32),
        pltpu.VMEM((SLOTS * EMBED, 128), jnp.float32),
        pltpu.VMEM((SLOTS * EMBED, 128), jnp.float32),
        pltpu.VMEM((B_PER_W,), jnp.float32),
        pltpu.SemaphoreType.DMA,
        pltpu.SemaphoreType.DMA,
        pltpu.SemaphoreType.DMA,
    ],
)
def _sc_dot(user_tt, item_tt, uid_h, iid_h, out_h,
            uids, iids, ublk, iblk, outv, sem_id, sem0, sem1):
    wid = lax.axis_index("s") * NUM_CORES + lax.axis_index("c")
    base = wid * B_PER_W

    pltpu.async_copy(
        uid_h.at[pl.ds(base, B_PER_W)], uids.at[pl.ds(0, B_PER_W)],
        sem_id).wait()
    pltpu.async_copy(
        iid_h.at[pl.ds(base, B_PER_W)], iids.at[pl.ds(0, B_PER_W)],
        sem_id).wait()

    lane_iota = lax.iota(jnp.int32, LANES)

    def fire_wave(w, phase_sem, phase):
        uvec = uids[pl.ds(w * WAVE, LANES)]
        ivec = iids[pl.ds(w * WAVE, LANES)]
        copies = []
        for k in range(WAVE):
            slot = phase * WAVE + k
            ual = pl.multiple_of((uvec[k] >> 7) << 7, 128)
            ial = pl.multiple_of((ivec[k] >> 7) << 7, 128)
            for e in range(EMBED // 8):
                copies.append(pltpu.async_copy(
                    user_tt.at[pl.ds(e * 8, 8), pl.ds(ual, 128)],
                    ublk.at[pl.ds(slot * EMBED + e * 8, 8)], phase_sem))
                copies.append(pltpu.async_copy(
                    item_tt.at[pl.ds(e * 8, 8), pl.ds(ial, 128)],
                    iblk.at[pl.ds(slot * EMBED + e * 8, 8)], phase_sem))
        return copies, uvec, ivec

    fire_wave(0, sem0, 0)

    def pair_body(p, acc):
        for phase in range(2):
            w = p * 2 + phase
            psem = (sem0, sem1)[phase]
            nsem = (sem0, sem1)[1 - phase]

            @pl.when(w + 1 < NWAVES)
            def _():
                fire_wave(w + 1, nsem, 1 - phase)

            uvec = uids[pl.ds(w * WAVE, LANES)]
            ivec = iids[pl.ds(w * WAVE, LANES)]
            for k in range(WAVE):
                slot = phase * WAVE + k
                pltpu.make_async_copy(
                    user_tt.at[:, pl.ds(0, 128)],
                    ublk.at[pl.ds(slot * EMBED, EMBED)], psem).wait()
                pltpu.make_async_copy(
                    item_tt.at[:, pl.ds(0, 128)],
                    iblk.at[pl.ds(slot * EMBED, EMBED)], psem).wait()
            for k in range(WAVE):
                slot = phase * WAVE + k
                kk = (w % 4) * WAVE + k
                uc = jnp.full((LANES,), uvec[k] & 127, jnp.int32)
                ic = jnp.full((LANES,), ivec[k] & 127, jnp.int32)
                r0 = lane_iota + (slot * EMBED)
                r1 = r0 + LANES
                u0 = plsc.load_gather(ublk, [r0, uc])
                u1 = plsc.load_gather(ublk, [r1, uc])
                v0 = plsc.load_gather(iblk, [r0, ic])
                v1 = plsc.load_gather(iblk, [r1, ic])
                s = jnp.sum(u0 * v0 + u1 * v1)
                acc = jnp.where(lane_iota == kk, s, acc)

            @pl.when(w % 4 == 3)
            def _():
                outv[pl.ds((w // 4) * LANES, LANES)] = acc
        return acc

    lax.fori_loop(0, NWAVES // 2, pair_body, jnp.zeros((LANES,), jnp.float32))

    pltpu.sync_copy(outv, out_h.at[pl.ds(base, B_PER_W)])


def kernel(user_table, item_table, user_ids, item_ids):
    utt = user_table.T
    itt = item_table.T
    uid = user_ids.reshape(BATCH)
    iid = item_ids.reshape(BATCH)
    out = _sc_dot(utt, itt, uid, iid)
    return out.reshape(BATCH, 1)


# speedup vs baseline: 1.0061x; 1.0061x over previous
import functools

import jax
import jax.numpy as jnp
from jax import lax
from jax.experimental import pallas as pl
from jax.experimental.pallas import tpu as pltpu
from jax.experimental.pallas import tpu_sc as plsc

BATCH = 16384
VOCAB_SIZE = 1000000
EMBED = 32
LANES = 16
NUM_CORES = 2
NUM_SUBCORES = 16
NUM_WORKERS = NUM_CORES * NUM_SUBCORES
B_PER_W = BATCH // NUM_WORKERS
WAVE = 4
NWAVES = B_PER_W // WAVE
SLOTS = 2 * WAVE


@functools.partial(
    pl.kernel,
    mesh=plsc.VectorSubcoreMesh(core_axis_name="c", subcore_axis_name="s"),
    out_type=jax.ShapeDtypeStruct((BATCH,), jnp.float32),
    compiler_params=pltpu.CompilerParams(needs_layout_passes=False),
    scratch_types=[
        pltpu.VMEM((B_PER_W + LANES,), jnp.int32),
        pltpu.VMEM((B_PER_W + LANES,), jnp.int32),
        pltpu.VMEM((SLOTS * EMBED, 128), jnp.float32),
        pltpu.VMEM((SLOTS * EMBED, 128), jnp.float32),
        pltpu.VMEM((B_PER_W,), jnp.float32),
        pltpu.SemaphoreType.DMA,
        pltpu.SemaphoreType.DMA,
        pltpu.SemaphoreType.DMA,
    ],
)
def _sc_dot(user_tt, item_tt, uid_h, iid_h, out_h,
            uids, iids, ublk, iblk, outv, sem_id, sem0, sem1):
    wid = lax.axis_index("s") * NUM_CORES + lax.axis_index("c")
    base = wid * B_PER_W

    pltpu.async_copy(
        uid_h.at[pl.ds(base, B_PER_W)], uids.at[pl.ds(0, B_PER_W)],
        sem_id).wait()
    pltpu.async_copy(
        iid_h.at[pl.ds(base, B_PER_W)], iids.at[pl.ds(0, B_PER_W)],
        sem_id).wait()

    lane_iota = lax.iota(jnp.int32, LANES)

    def fire_wave(w, phase_sem, phase):
        uvec = uids[pl.ds(w * WAVE, LANES)]
        ivec = iids[pl.ds(w * WAVE, LANES)]
        copies = []
        for k in range(WAVE):
            slot = phase * WAVE + k
            ual = pl.multiple_of((uvec[k] >> 7) << 7, 128)
            ial = pl.multiple_of((ivec[k] >> 7) << 7, 128)
            copies.append(pltpu.async_copy(
                user_tt.at[:, pl.ds(ual, 128)],
                ublk.at[pl.ds(slot * EMBED, EMBED)], phase_sem))
            copies.append(pltpu.async_copy(
                item_tt.at[:, pl.ds(ial, 128)],
                iblk.at[pl.ds(slot * EMBED, EMBED)], phase_sem))
        return copies, uvec, ivec

    fire_wave(0, sem0, 0)

    def pair_body(p, acc):
        for phase in range(2):
            w = p * 2 + phase
            psem = (sem0, sem1)[phase]
            nsem = (sem0, sem1)[1 - phase]

            @pl.when(w + 1 < NWAVES)
            def _():
                fire_wave(w + 1, nsem, 1 - phase)

            uvec = uids[pl.ds(w * WAVE, LANES)]
            ivec = iids[pl.ds(w * WAVE, LANES)]
            for k in range(WAVE):
                slot = phase * WAVE + k
                pltpu.make_async_copy(
                    user_tt.at[:, pl.ds(0, 128)],
                    ublk.at[pl.ds(slot * EMBED, EMBED)], psem).wait()
                pltpu.make_async_copy(
                    item_tt.at[:, pl.ds(0, 128)],
                    iblk.at[pl.ds(slot * EMBED, EMBED)], psem).wait()
            for k in range(WAVE):
                slot = phase * WAVE + k
                kk = (w % 4) * WAVE + k
                uc = jnp.full((LANES,), uvec[k] & 127, jnp.int32)
                ic = jnp.full((LANES,), ivec[k] & 127, jnp.int32)
                r0 = lane_iota + (slot * EMBED)
                r1 = r0 + LANES
                u0 = plsc.load_gather(ublk, [r0, uc])
                u1 = plsc.load_gather(ublk, [r1, uc])
                v0 = plsc.load_gather(iblk, [r0, ic])
                v1 = plsc.load_gather(iblk, [r1, ic])
                s = jnp.sum(u0 * v0 + u1 * v1)
                acc = jnp.where(lane_iota == kk, s, acc)

            @pl.when(w % 4 == 3)
            def _():
                outv[pl.ds((w // 4) * LANES, LANES)] = acc
        return acc

    lax.fori_loop(0, NWAVES // 2, pair_body, jnp.zeros((LANES,), jnp.float32))

    pltpu.sync_copy(outv, out_h.at[pl.ds(base, B_PER_W)])


def kernel(user_table, item_table, user_ids, item_ids):
    utt = user_table.T
    itt = item_table.T
    uid = user_ids.reshape(BATCH)
    iid = item_ids.reshape(BATCH)
    out = _sc_dot(utt, itt, uid, iid)
    return out.reshape(BATCH, 1)


# speedup vs baseline: 1.1022x; 1.0956x over previous
import functools

import jax
import jax.numpy as jnp
from jax import lax
from jax.experimental import pallas as pl
from jax.experimental.pallas import tpu as pltpu
from jax.experimental.pallas import tpu_sc as plsc

BATCH = 16384
VOCAB_SIZE = 1000000
EMBED = 32
LANES = 16
NUM_CORES = 2
NUM_SUBCORES = 16
NUM_WORKERS = NUM_CORES * NUM_SUBCORES
B_PER_W = BATCH // NUM_WORKERS
WAVE = 2
NWAVES = B_PER_W // WAVE
PHASES = 4
SLOTS = PHASES * WAVE


@functools.partial(
    pl.kernel,
    mesh=plsc.VectorSubcoreMesh(core_axis_name="c", subcore_axis_name="s"),
    out_type=jax.ShapeDtypeStruct((BATCH,), jnp.float32),
    compiler_params=pltpu.CompilerParams(needs_layout_passes=False),
    scratch_types=[
        pltpu.VMEM((B_PER_W + LANES,), jnp.int32),
        pltpu.VMEM((B_PER_W + LANES,), jnp.int32),
        pltpu.VMEM((SLOTS * EMBED, 128), jnp.float32),
        pltpu.VMEM((SLOTS * EMBED, 128), jnp.float32),
        pltpu.VMEM((B_PER_W,), jnp.float32),
        pltpu.SemaphoreType.DMA,
        pltpu.SemaphoreType.DMA,
        pltpu.SemaphoreType.DMA,
        pltpu.SemaphoreType.DMA,
        pltpu.SemaphoreType.DMA,
    ],
)
def _sc_dot(user_tt, item_tt, uid_h, iid_h, out_h,
            uids, iids, ublk, iblk, outv, sem_id, sem0, sem1, sem2, sem3):
    wid = lax.axis_index("s") * NUM_CORES + lax.axis_index("c")
    base = wid * B_PER_W

    pltpu.async_copy(
        uid_h.at[pl.ds(base, B_PER_W)], uids.at[pl.ds(0, B_PER_W)],
        sem_id).wait()
    pltpu.async_copy(
        iid_h.at[pl.ds(base, B_PER_W)], iids.at[pl.ds(0, B_PER_W)],
        sem_id).wait()

    lane_iota = lax.iota(jnp.int32, LANES)

    def fire_wave(w, phase_sem, phase):
        uvec = uids[pl.ds(w * WAVE, LANES)]
        ivec = iids[pl.ds(w * WAVE, LANES)]
        copies = []
        for k in range(WAVE):
            slot = phase * WAVE + k
            ual = pl.multiple_of((uvec[k] >> 7) << 7, 128)
            ial = pl.multiple_of((ivec[k] >> 7) << 7, 128)
            copies.append(pltpu.async_copy(
                user_tt.at[:, pl.ds(ual, 128)],
                ublk.at[pl.ds(slot * EMBED, EMBED)], phase_sem))
            copies.append(pltpu.async_copy(
                item_tt.at[:, pl.ds(ial, 128)],
                iblk.at[pl.ds(slot * EMBED, EMBED)], phase_sem))
        return copies, uvec, ivec

    sems = (sem0, sem1, sem2, sem3)

    for w0 in range(PHASES - 1):
        fire_wave(w0, sems[w0], w0)

    def group_body(p, acc):
        for phase in range(PHASES):
            w = p * PHASES + phase

            @pl.when(w + PHASES - 1 < NWAVES)
            def _():
                fire_wave(w + PHASES - 1, sems[(phase + PHASES - 1) % PHASES],
                          (phase + PHASES - 1) % PHASES)

            uvec = uids[pl.ds(w * WAVE, LANES)]
            ivec = iids[pl.ds(w * WAVE, LANES)]
            for k in range(WAVE):
                slot = phase * WAVE + k
                pltpu.make_async_copy(
                    user_tt.at[:, pl.ds(0, 128)],
                    ublk.at[pl.ds(slot * EMBED, EMBED)], sems[phase]).wait()
                pltpu.make_async_copy(
                    item_tt.at[:, pl.ds(0, 128)],
                    iblk.at[pl.ds(slot * EMBED, EMBED)], sems[phase]).wait()
            for k in range(WAVE):
                slot = phase * WAVE + k
                kk = (w % (LANES // WAVE)) * WAVE + k
                uc = jnp.full((LANES,), uvec[k] & 127, jnp.int32)
                ic = jnp.full((LANES,), ivec[k] & 127, jnp.int32)
                r0 = lane_iota + (slot * EMBED)
                r1 = r0 + LANES
                u0 = plsc.load_gather(ublk, [r0, uc])
                u1 = plsc.load_gather(ublk, [r1, uc])
                v0 = plsc.load_gather(iblk, [r0, ic])
                v1 = plsc.load_gather(iblk, [r1, ic])
                s = jnp.sum(u0 * v0 + u1 * v1)
                acc = jnp.where(lane_iota == kk, s, acc)

            @pl.when(w % (LANES // WAVE) == LANES // WAVE - 1)
            def _():
                outv[pl.ds((w // (LANES // WAVE)) * LANES, LANES)] = acc
        return acc

    lax.fori_loop(0, NWAVES // PHASES, group_body,
                  jnp.zeros((LANES,), jnp.float32))

    pltpu.sync_copy(outv, out_h.at[pl.ds(base, B_PER_W)])


def kernel(user_table, item_table, user_ids, item_ids):
    utt = user_table.T
    itt = item_table.T
    uid = user_ids.reshape(BATCH)
    iid = item_ids.reshape(BATCH)
    out = _sc_dot(utt, itt, uid, iid)
    return out.reshape(BATCH, 1)


# speedup vs baseline: 1.1886x; 1.0783x over previous
import functools

import jax
import jax.numpy as jnp
from jax import lax
from jax.experimental import pallas as pl
from jax.experimental.pallas import tpu as pltpu
from jax.experimental.pallas import tpu_sc as plsc

BATCH = 16384
VOCAB_SIZE = 1000000
EMBED = 32
LANES = 16
NUM_CORES = 2
NUM_SUBCORES = 16
NUM_WORKERS = NUM_CORES * NUM_SUBCORES
B_PER_W = BATCH // NUM_WORKERS
WAVE = 1
NWAVES = B_PER_W // WAVE
PHASES = 8
SLOTS = PHASES * WAVE


@functools.partial(
    pl.kernel,
    mesh=plsc.VectorSubcoreMesh(core_axis_name="c", subcore_axis_name="s"),
    out_type=jax.ShapeDtypeStruct((BATCH,), jnp.float32),
    compiler_params=pltpu.CompilerParams(needs_layout_passes=False),
    scratch_types=[
        pltpu.VMEM((B_PER_W + LANES,), jnp.int32),
        pltpu.VMEM((B_PER_W + LANES,), jnp.int32),
        pltpu.VMEM((SLOTS * EMBED, 128), jnp.float32),
        pltpu.VMEM((SLOTS * EMBED, 128), jnp.float32),
        pltpu.VMEM((B_PER_W,), jnp.float32),
        pltpu.SemaphoreType.DMA,
        pltpu.SemaphoreType.DMA,
        pltpu.SemaphoreType.DMA,
        pltpu.SemaphoreType.DMA,
        pltpu.SemaphoreType.DMA,
        pltpu.SemaphoreType.DMA,
        pltpu.SemaphoreType.DMA,
        pltpu.SemaphoreType.DMA,
        pltpu.SemaphoreType.DMA,
    ],
)
def _sc_dot(user_tt, item_tt, uid_h, iid_h, out_h,
            uids, iids, ublk, iblk, outv, sem_id,
            sem0, sem1, sem2, sem3, sem4, sem5, sem6, sem7):
    wid = lax.axis_index("s") * NUM_CORES + lax.axis_index("c")
    base = wid * B_PER_W

    pltpu.async_copy(
        uid_h.at[pl.ds(base, B_PER_W)], uids.at[pl.ds(0, B_PER_W)],
        sem_id).wait()
    pltpu.async_copy(
        iid_h.at[pl.ds(base, B_PER_W)], iids.at[pl.ds(0, B_PER_W)],
        sem_id).wait()

    lane_iota = lax.iota(jnp.int32, LANES)

    def fire_wave(w, phase_sem, phase):
        uvec = uids[pl.ds(w * WAVE, LANES)]
        ivec = iids[pl.ds(w * WAVE, LANES)]
        copies = []
        for k in range(WAVE):
            slot = phase * WAVE + k
            ual = pl.multiple_of((uvec[k] >> 7) << 7, 128)
            ial = pl.multiple_of((ivec[k] >> 7) << 7, 128)
            copies.append(pltpu.async_copy(
                user_tt.at[:, pl.ds(ual, 128)],
                ublk.at[pl.ds(slot * EMBED, EMBED)], phase_sem))
            copies.append(pltpu.async_copy(
                item_tt.at[:, pl.ds(ial, 128)],
                iblk.at[pl.ds(slot * EMBED, EMBED)], phase_sem))
        return copies, uvec, ivec

    sems = (sem0, sem1, sem2, sem3, sem4, sem5, sem6, sem7)

    for w0 in range(PHASES - 1):
        fire_wave(w0, sems[w0], w0)

    def group_body(p, acc):
        for phase in range(PHASES):
            w = p * PHASES + phase

            @pl.when(w + PHASES - 1 < NWAVES)
            def _():
                fire_wave(w + PHASES - 1, sems[(phase + PHASES - 1) % PHASES],
                          (phase + PHASES - 1) % PHASES)

            uvec = uids[pl.ds(w * WAVE, LANES)]
            ivec = iids[pl.ds(w * WAVE, LANES)]
            for k in range(WAVE):
                slot = phase * WAVE + k
                pltpu.make_async_copy(
                    user_tt.at[:, pl.ds(0, 128)],
                    ublk.at[pl.ds(slot * EMBED, EMBED)], sems[phase]).wait()
                pltpu.make_async_copy(
                    item_tt.at[:, pl.ds(0, 128)],
                    iblk.at[pl.ds(slot * EMBED, EMBED)], sems[phase]).wait()
            for k in range(WAVE):
                slot = phase * WAVE + k
                kk = (w % (LANES // WAVE)) * WAVE + k
                uc = jnp.full((LANES,), uvec[k] & 127, jnp.int32)
                ic = jnp.full((LANES,), ivec[k] & 127, jnp.int32)
                r0 = lane_iota + (slot * EMBED)
                r1 = r0 + LANES
                u0 = plsc.load_gather(ublk, [r0, uc])
                u1 = plsc.load_gather(ublk, [r1, uc])
                v0 = plsc.load_gather(iblk, [r0, ic])
                v1 = plsc.load_gather(iblk, [r1, ic])
                s = jnp.sum(u0 * v0 + u1 * v1)
                acc = jnp.where(lane_iota == kk, s, acc)

            @pl.when(w % (LANES // WAVE) == LANES // WAVE - 1)
            def _():
                outv[pl.ds((w // (LANES // WAVE)) * LANES, LANES)] = acc
        return acc

    lax.fori_loop(0, NWAVES // PHASES, group_body,
                  jnp.zeros((LANES,), jnp.float32))

    pltpu.sync_copy(outv, out_h.at[pl.ds(base, B_PER_W)])


def kernel(user_table, item_table, user_ids, item_ids):
    utt = user_table.T
    itt = item_table.T
    uid = user_ids.reshape(BATCH)
    iid = item_ids.reshape(BATCH)
    out = _sc_dot(utt, itt, uid, iid)
    return out.reshape(BATCH, 1)


# speedup vs baseline: 1.1996x; 1.0092x over previous
import functools

import jax
import jax.numpy as jnp
from jax import lax
from jax.experimental import pallas as pl
from jax.experimental.pallas import tpu as pltpu
from jax.experimental.pallas import tpu_sc as plsc

BATCH = 16384
VOCAB_SIZE = 1000000
EMBED = 32
LANES = 16
NUM_CORES = 2
NUM_SUBCORES = 16
NUM_WORKERS = NUM_CORES * NUM_SUBCORES
B_PER_W = BATCH // NUM_WORKERS
WAVE = 1
NWAVES = B_PER_W // WAVE
PHASES = 15
SLOTS = PHASES * WAVE
NTAIL = NWAVES % PHASES
NGROUPS = NWAVES // PHASES


@functools.partial(
    pl.kernel,
    mesh=plsc.VectorSubcoreMesh(core_axis_name="c", subcore_axis_name="s"),
    out_type=jax.ShapeDtypeStruct((BATCH,), jnp.float32),
    compiler_params=pltpu.CompilerParams(needs_layout_passes=False),
    scratch_types=[
        pltpu.VMEM((B_PER_W + LANES,), jnp.int32),
        pltpu.VMEM((B_PER_W + LANES,), jnp.int32),
        pltpu.VMEM((SLOTS * EMBED, 128), jnp.float32),
        pltpu.VMEM((SLOTS * EMBED, 128), jnp.float32),
        pltpu.VMEM((B_PER_W,), jnp.float32),
    ] + [pltpu.SemaphoreType.DMA] * (1 + PHASES),
)
def _sc_dot(user_tt, item_tt, uid_h, iid_h, out_h,
            uids, iids, ublk, iblk, outv, sem_id, *sems):
    wid = lax.axis_index("s") * NUM_CORES + lax.axis_index("c")
    base = wid * B_PER_W

    pltpu.async_copy(
        uid_h.at[pl.ds(base, B_PER_W)], uids.at[pl.ds(0, B_PER_W)],
        sem_id).wait()
    pltpu.async_copy(
        iid_h.at[pl.ds(base, B_PER_W)], iids.at[pl.ds(0, B_PER_W)],
        sem_id).wait()

    lane_iota = lax.iota(jnp.int32, LANES)

    def fire_wave(w, phase_sem, phase):
        uvec = uids[pl.ds(w * WAVE, LANES)]
        ivec = iids[pl.ds(w * WAVE, LANES)]
        copies = []
        for k in range(WAVE):
            slot = phase * WAVE + k
            ual = pl.multiple_of((uvec[k] >> 7) << 7, 128)
            ial = pl.multiple_of((ivec[k] >> 7) << 7, 128)
            copies.append(pltpu.async_copy(
                user_tt.at[:, pl.ds(ual, 128)],
                ublk.at[pl.ds(slot * EMBED, EMBED)], phase_sem))
            copies.append(pltpu.async_copy(
                item_tt.at[:, pl.ds(ial, 128)],
                iblk.at[pl.ds(slot * EMBED, EMBED)], phase_sem))
        return copies, uvec, ivec

    for w0 in range(PHASES - 1):
        fire_wave(w0, sems[w0], w0)

    def process_wave(w, phase, acc, do_fire):
        if do_fire:
            @pl.when(w + PHASES - 1 < NWAVES)
            def _():
                fire_wave(w + PHASES - 1, sems[(phase + PHASES - 1) % PHASES],
                          (phase + PHASES - 1) % PHASES)

        uvec = uids[pl.ds(w * WAVE, LANES)]
        ivec = iids[pl.ds(w * WAVE, LANES)]
        for k in range(WAVE):
            slot = phase * WAVE + k
            pltpu.make_async_copy(
                user_tt.at[:, pl.ds(0, 128)],
                ublk.at[pl.ds(slot * EMBED, EMBED)], sems[phase]).wait()
            pltpu.make_async_copy(
                item_tt.at[:, pl.ds(0, 128)],
                iblk.at[pl.ds(slot * EMBED, EMBED)], sems[phase]).wait()
        for k in range(WAVE):
            slot = phase * WAVE + k
            kk = (w % (LANES // WAVE)) * WAVE + k
            uc = jnp.full((LANES,), uvec[k] & 127, jnp.int32)
            ic = jnp.full((LANES,), ivec[k] & 127, jnp.int32)
            r0 = lane_iota + (slot * EMBED)
            r1 = r0 + LANES
            u0 = plsc.load_gather(ublk, [r0, uc])
            u1 = plsc.load_gather(ublk, [r1, uc])
            v0 = plsc.load_gather(iblk, [r0, ic])
            v1 = plsc.load_gather(iblk, [r1, ic])
            s = jnp.sum(u0 * v0 + u1 * v1)
            acc = jnp.where(lane_iota == kk, s, acc)

        @pl.when(w % (LANES // WAVE) == LANES // WAVE - 1)
        def _():
            outv[pl.ds((w // (LANES // WAVE)) * LANES, LANES)] = acc
        return acc

    def group_body(p, acc):
        for phase in range(PHASES):
            acc = process_wave(p * PHASES + phase, phase, acc, True)
        return acc

    acc = lax.fori_loop(0, NGROUPS, group_body,
                        jnp.zeros((LANES,), jnp.float32))
    for t in range(NTAIL):
        w = NGROUPS * PHASES + t
        acc = process_wave(w, w % PHASES, acc, False)

    pltpu.sync_copy(outv, out_h.at[pl.ds(base, B_PER_W)])


def kernel(user_table, item_table, user_ids, item_ids):
    utt = user_table.T
    itt = item_table.T
    uid = user_ids.reshape(BATCH)
    iid = item_ids.reshape(BATCH)
    out = _sc_dot(utt, itt, uid, iid)
    return out.reshape(BATCH, 1)
